# baseline (speedup 1.0000x reference)
import functools

import jax
import jax.numpy as jnp
from jax import lax
from jax.experimental import pallas as pl
from jax.experimental.pallas import tpu as pltpu
from jax.experimental.pallas import tpu_sc as plsc

_VOCAB = 1000000
_D = 64
_B = 4096
_S = 200

_NC = 2
_NS = 16
_NW = _NC * _NS
_BPW = _B // _NW


def _pool_body(x_hbm, table_hbm, out_hbm, x_v, acc_v, sem):
    wid = lax.axis_index("s") * _NC + lax.axis_index("c")
    pltpu.sync_copy(x_hbm.at[wid], x_v)
    pltpu.async_copy(table_hbm.at[x_v.at[0]], acc_v, sem).wait()

    def body(s, carry):
        pltpu.async_copy(table_hbm.at[x_v.at[s]], acc_v, sem, add=True).wait()
        return carry

    lax.fori_loop(1, _S, body, 0)
    pltpu.sync_copy(acc_v, out_hbm.at[pl.ds(wid * _BPW, _BPW)])


@jax.jit
def _pool(x_r, table):
    mesh = plsc.VectorSubcoreMesh(
        core_axis_name="c", subcore_axis_name="s", num_cores=_NC,
        num_subcores=_NS)
    return pl.kernel(
        _pool_body,
        out_type=jax.ShapeDtypeStruct((_B, _D), jnp.float32),
        mesh=mesh,
        scratch_types=[
            pltpu.VMEM((_S, _BPW), jnp.int32),
            pltpu.VMEM((_BPW, _D), jnp.float32),
            pltpu.SemaphoreType.DMA,
        ],
        compiler_params=pltpu.CompilerParams(use_tc_tiling_on_sc=False),
    )(x_r, table)


def _fc_body(acc_ref, wfc_t_ref, bfc_ref, wfc1_t_ref, bfc1_ref, out_ref):
    h = jnp.dot(acc_ref[...], wfc_t_ref[...],
                preferred_element_type=jnp.float32) + bfc_ref[...]
    h = jnp.maximum(h, 0.0)
    out_ref[...] = jnp.dot(h, wfc1_t_ref[...],
                           preferred_element_type=jnp.float32) + bfc1_ref[...]


@jax.jit
def _fc(pooled, wfc_t, bfc, wfc1_t, bfc1):
    nc = wfc1_t.shape[1]
    return pl.pallas_call(
        _fc_body,
        out_shape=jax.ShapeDtypeStruct((_B, nc), jnp.float32),
    )(pooled, wfc_t, bfc, wfc1_t, bfc1)


def kernel(x, table, W_fc, b_fc, W_fc1, b_fc1):
    x_r = jnp.swapaxes(x.astype(jnp.int32).reshape(_NW, _BPW, _S), 1, 2)
    pooled = _pool(x_r, table)
    return _fc(pooled, W_fc.T, b_fc.reshape(1, -1), W_fc1.T,
               b_fc1.reshape(1, -1))


# speedup vs baseline: 1.1563x; 1.1563x over previous
import jax
import jax.numpy as jnp
from jax import lax
from jax.experimental import pallas as pl
from jax.experimental.pallas import tpu as pltpu
from jax.experimental.pallas import tpu_sc as plsc

_VOCAB = 1000000
_D = 64
_B = 4096
_S = 200

_NC = 2
_NS = 16
_NW = _NC * _NS
_BPW = _B // _NW
_W = 8


def _pool_body(x_hbm, table_hbm, out_hbm, x_v, xt_v, acc_v, sems):
    wid = lax.axis_index("s") * _NC + lax.axis_index("c")
    pltpu.sync_copy(x_hbm.at[wid], x_v)

    zeros = jnp.zeros((16,), jnp.float32)

    def zero_row(b, c):
        acc_v[b, pl.ds(0, 16)] = zeros
        acc_v[b, pl.ds(16, 16)] = zeros
        acc_v[b, pl.ds(32, 16)] = zeros
        acc_v[b, pl.ds(48, 16)] = zeros
        return c

    lax.fori_loop(0, _BPW, zero_row, 0)

    iota = lax.iota(jnp.int32, 16)

    def transpose_row(s, c):
        for ch in range(_BPW // 16):
            idx = (iota + (ch * 16)) * _S + s
            col = plsc.load_gather(x_v, [idx])
            xt_v[s, pl.ds(ch * 16, 16)] = col
        return c

    lax.fori_loop(0, _S, transpose_row, 0)

    def fire(s, slot):
        pltpu.async_copy(table_hbm.at[xt_v.at[s]], acc_v, sems.at[slot],
                         add=True)

    def ring_wait(slot):
        pltpu.make_async_copy(table_hbm.at[xt_v.at[0]], acc_v,
                              sems.at[slot]).wait()

    for j in range(_W):
        fire(j, j)

    def steady(i, c):
        slot = lax.rem(i, _W)
        ring_wait(slot)
        fire(i + _W, slot)
        return c

    lax.fori_loop(0, _S - _W, steady, 0)
    for j in range(_W):
        ring_wait(j)

    pltpu.sync_copy(acc_v, out_hbm.at[pl.ds(wid * _BPW, _BPW)])


@jax.jit
def _pool(x_r, table):
    mesh = plsc.VectorSubcoreMesh(
        core_axis_name="c", subcore_axis_name="s", num_cores=_NC,
        num_subcores=_NS)
    return pl.kernel(
        _pool_body,
        out_type=jax.ShapeDtypeStruct((_B, _D), jnp.float32),
        mesh=mesh,
        scratch_types=[
            pltpu.VMEM((_BPW * _S,), jnp.int32),
            pltpu.VMEM((_S, _BPW), jnp.int32),
            pltpu.VMEM((_BPW, _D), jnp.float32),
            pltpu.SemaphoreType.DMA((_W,)),
        ],
        compiler_params=pltpu.CompilerParams(use_tc_tiling_on_sc=False,
                                             needs_layout_passes=False),
    )(x_r, table)


def _fc_body(acc_ref, wfc_t_ref, bfc_ref, wfc1_t_ref, bfc1_ref, out_ref):
    h = jnp.dot(acc_ref[...], wfc_t_ref[...],
                preferred_element_type=jnp.float32) + bfc_ref[...]
    h = jnp.maximum(h, 0.0)
    out_ref[...] = jnp.dot(h, wfc1_t_ref[...],
                           preferred_element_type=jnp.float32) + bfc1_ref[...]


@jax.jit
def _fc(pooled, wfc_t, bfc, wfc1_t, bfc1):
    nc = wfc1_t.shape[1]
    return pl.pallas_call(
        _fc_body,
        out_shape=jax.ShapeDtypeStruct((_B, nc), jnp.float32),
    )(pooled, wfc_t, bfc, wfc1_t, bfc1)


def kernel(x, table, W_fc, b_fc, W_fc1, b_fc1):
    x_r = x.astype(jnp.int32).reshape(_NW, _BPW * _S)
    pooled = _pool(x_r, table)
    return _fc(pooled, W_fc.T, b_fc.reshape(1, -1), W_fc1.T,
               b_fc1.reshape(1, -1))
